# baseline (speedup 1.0000x reference)
import functools

import jax
import jax.numpy as jnp
from jax import lax
from jax.experimental import pallas as pl
from jax.experimental.pallas import tpu as pltpu
from jax.experimental.pallas import tpu_sc as plsc

NUM_CORES = 2
NUM_SUBCORES = 16
NUM_WORKERS = NUM_CORES * NUM_SUBCORES
LANES = 16
BATCH_N = 16384
FEAT = 64
NUM_ROWS = 1000000
ROWS_PER_W = BATCH_N // NUM_WORKERS
CHUNK = 128
NCHUNK = ROWS_PER_W // CHUNK


def _body(user_hbm, item_hbm, uflat_hbm, iflat_hbm, out_hbm,
          uidx_v, iidx_v, ubuf_v, ibuf_v, us_v, is_v, out_v, sem):
    wid = lax.axis_index("s") * NUM_CORES + lax.axis_index("c")
    base = wid * ROWS_PER_W

    pltpu.sync_copy(user_hbm.at[pl.ds(base, ROWS_PER_W)], uidx_v)
    pltpu.sync_copy(item_hbm.at[pl.ds(base, ROWS_PER_W)], iidx_v)

    for j in range(NCHUNK):
        def build(c, _):
            off = c * NUM_ROWS
            for v in range(CHUNK // LANES):
                sl = pl.ds(j * CHUNK + v * LANES, LANES)
                dsl = pl.ds(v * LANES, LANES)
                ubuf_v[c, dsl] = uidx_v[sl] + off
                ibuf_v[c, dsl] = iidx_v[sl] + off
            return ()

        lax.fori_loop(0, FEAT, build, ())

        def fire(c, _):
            pltpu.async_copy(uflat_hbm.at[ubuf_v.at[c]],
                             us_v.at[c, pl.ds(j * CHUNK, CHUNK)], sem)
            pltpu.async_copy(iflat_hbm.at[ibuf_v.at[c]],
                             is_v.at[c, pl.ds(j * CHUNK, CHUNK)], sem)
            return ()

        lax.fori_loop(0, FEAT, fire, ())

        def drain(c, _):
            pltpu.make_async_copy(
                uflat_hbm.at[pl.ds(0, CHUNK)],
                us_v.at[c, pl.ds(j * CHUNK, CHUNK)], sem).wait()
            pltpu.make_async_copy(
                iflat_hbm.at[pl.ds(0, CHUNK)],
                is_v.at[c, pl.ds(j * CHUNK, CHUNK)], sem).wait()
            return ()

        lax.fori_loop(0, FEAT, drain, ())

    def grp_body(g, _):
        sl = pl.ds(g * LANES, LANES)
        acc = jnp.zeros((LANES,), jnp.float32)
        accs = [jnp.zeros((LANES,), jnp.float32) for _ in range(4)]
        for c in range(FEAT):
            accs[c % 4] = accs[c % 4] + us_v[c, sl] * is_v[c, sl]
        out_v[sl] = (accs[0] + accs[1]) + (accs[2] + accs[3])
        return ()

    lax.fori_loop(0, ROWS_PER_W // LANES, grp_body, ())

    pltpu.sync_copy(out_v, out_hbm.at[pl.ds(base, ROWS_PER_W)])


@jax.jit
def kernel(user, item, users_emb, items_emb):
    uflat = users_emb.T.reshape(-1)
    iflat = items_emb.T.reshape(-1)
    mesh = plsc.VectorSubcoreMesh(core_axis_name="c", subcore_axis_name="s")
    k = pl.kernel(
        _body,
        out_type=jax.ShapeDtypeStruct((BATCH_N,), jnp.float32),
        mesh=mesh,
        scratch_types=[
            pltpu.VMEM((ROWS_PER_W,), jnp.int32),
            pltpu.VMEM((ROWS_PER_W,), jnp.int32),
            pltpu.VMEM((FEAT, CHUNK), jnp.int32),
            pltpu.VMEM((FEAT, CHUNK), jnp.int32),
            pltpu.VMEM((FEAT, ROWS_PER_W), jnp.float32),
            pltpu.VMEM((FEAT, ROWS_PER_W), jnp.float32),
            pltpu.VMEM((ROWS_PER_W,), jnp.float32),
            pltpu.SemaphoreType.DMA,
        ],
        compiler_params=pltpu.CompilerParams(
            needs_layout_passes=False, use_tc_tiling_on_sc=False),
    )
    return k(user.astype(jnp.int32), item.astype(jnp.int32), uflat, iflat)
